# baseline (speedup 1.0000x reference)
import functools

import jax
import jax.numpy as jnp
from jax import lax
from jax.experimental import pallas as pl
from jax.experimental.pallas import tpu as pltpu
from jax.experimental.pallas import tpu_sc as plsc

N = 10000
E = 320000
D = 128
L = 16
G = 64

NC = 2
NS = 16
NW = NC * NS
CB = 112
CH = 92
EP = NW * CH * CB
NBUF = 3
NI = 4
NP = 10112
ROWS_PER_TILE = NP // NS


def _sc_agg_body(e_hbm, g_hbm, zeros_hbm, out_hbm, s_sh, rows_v, *rest):
    e_v = rest[:NI]
    gsem = rest[NI:NI + NBUF]
    csem = rest[NI + NBUF:NI + 2 * NBUF]
    isem = rest[NI + 2 * NBUF:]

    c = lax.axis_index("c")
    s = lax.axis_index("s")
    w = c * NS + s

    pltpu.sync_copy(zeros_hbm, s_sh.at[pl.ds(s * ROWS_PER_TILE, ROWS_PER_TILE)])

    def idx_issue(q, j):
        pltpu.async_copy(e_hbm.at[w * CH + j], e_v[q], isem[q])

    def idx_wait(q):
        pltpu.make_async_copy(e_hbm.at[0], e_v[0], isem[q]).wait()

    def gather_issue(b, q):
        pltpu.async_copy(g_hbm.at[e_v[q].at[0]], rows_v.at[b], gsem[b])

    def gather_wait(b):
        pltpu.make_async_copy(g_hbm.at[pl.ds(0, CB)], rows_v.at[b],
                              gsem[b]).wait()

    def scat_issue(b, q):
        pltpu.async_copy(rows_v.at[b], s_sh.at[e_v[q].at[1]], csem[b],
                         add=True)

    def scat_wait(b):
        pltpu.make_async_copy(g_hbm.at[pl.ds(0, CB)], rows_v.at[b],
                              csem[b]).wait()

    def body(j, jd=None, full_scat_wait=True, do_idx=True, do_gather=True):
        if jd is None:
            jd = j
        b = j % NBUF
        q = j % NI
        gather_wait(b)
        scat_issue(b, q)
        if full_scat_wait:
            scat_wait((j - 1) % NBUF)
        if do_idx:
            idx_issue((j + 3) % NI, jd + 3)
        if do_gather:
            idx_wait((j + 2) % NI)
            gather_issue((j + 2) % NBUF, (j + 2) % NI)

    for q in range(3):
        idx_issue(q, q)
    idx_wait(0)
    gather_issue(0, 0)
    idx_wait(1)
    gather_issue(1, 1)

    plsc.subcore_barrier()

    body(0, full_scat_wait=False)
    for j in range(1, 4):
        body(j)

    def step(i, carry):
        base = 4 + i * 12
        for u in range(12):
            body(4 + u, jd=base + u)
        return carry

    lax.fori_loop(0, (CH - 8) // 12, step, 0)

    body(CH - 4)
    body(CH - 3, do_idx=False)
    body(CH - 2, do_idx=False, do_gather=False)
    body(CH - 1, do_idx=False, do_gather=False)
    scat_wait((CH - 1) % NBUF)

    plsc.subcore_barrier()

    sl = pl.ds(s * ROWS_PER_TILE, ROWS_PER_TILE)
    pltpu.sync_copy(s_sh.at[sl], out_hbm.at[c].at[sl])


_sc_agg = functools.partial(
    pl.kernel,
    out_type=jax.ShapeDtypeStruct((NC, NP, D), jnp.float32),
    mesh=plsc.VectorSubcoreMesh(core_axis_name="c", subcore_axis_name="s",
                                num_cores=NC, num_subcores=NS),
    scratch_types=[
        pltpu.VMEM_SHARED((NP, D), jnp.float32),
        pltpu.VMEM((NBUF, CB, D), jnp.float32),
    ]
    + [pltpu.VMEM((2, CB), jnp.int32) for _ in range(NI)]
    + [pltpu.SemaphoreType.DMA] * (2 * NBUF + NI),
)(_sc_agg_body)


BR = 1000
GRID = N // BR


def _tc_first_body(s0_ref, s1_ref, x_ref, w_ref, g_ref, dinv_ref):
    deg = s0_ref[:, 0:1] + s1_ref[:, 0:1] + 1.0
    dinv = lax.rsqrt(deg)
    dinv_ref[...] = dinv
    xw = jnp.dot(x_ref[...], w_ref[...], preferred_element_type=jnp.float32)
    g_ref[...] = dinv * xw


def _tc_first(s0, s1, x, w0):
    return pl.pallas_call(
        _tc_first_body,
        grid=(GRID,),
        in_specs=[
            pl.BlockSpec((BR, D), lambda i: (i, 0)),
            pl.BlockSpec((BR, D), lambda i: (i, 0)),
            pl.BlockSpec((BR, D), lambda i: (i, 0)),
            pl.BlockSpec((D, D), lambda i: (0, 0)),
        ],
        out_specs=[
            pl.BlockSpec((BR, D), lambda i: (i, 0)),
            pl.BlockSpec((BR, 1), lambda i: (i, 0)),
        ],
        out_shape=[
            jax.ShapeDtypeStruct((N, D), jnp.float32),
            jax.ShapeDtypeStruct((N, 1), jnp.float32),
        ],
    )(s0, s1, x, w0)


def _tc_layer_body(s0_ref, s1_ref, gp_ref, dinv_ref, b_ref, w_ref, g_ref):
    dinv = dinv_ref[...]
    h = jnp.maximum(dinv * (s0_ref[...] + s1_ref[...] + gp_ref[...])
                    + b_ref[...], 0.0)
    hw = jnp.dot(h, w_ref[...], preferred_element_type=jnp.float32)
    g_ref[...] = dinv * hw


def _tc_layer(s0, s1, gp, dinv, b, w):
    return pl.pallas_call(
        _tc_layer_body,
        grid=(GRID,),
        in_specs=[
            pl.BlockSpec((BR, D), lambda i: (i, 0)),
            pl.BlockSpec((BR, D), lambda i: (i, 0)),
            pl.BlockSpec((BR, D), lambda i: (i, 0)),
            pl.BlockSpec((BR, 1), lambda i: (i, 0)),
            pl.BlockSpec((1, D), lambda i: (0, 0)),
            pl.BlockSpec((D, D), lambda i: (0, 0)),
        ],
        out_specs=pl.BlockSpec((BR, D), lambda i: (i, 0)),
        out_shape=jax.ShapeDtypeStruct((N, D), jnp.float32),
    )(s0, s1, gp, dinv, b, w)


def _tc_pool_body(s0_ref, s1_ref, gp_ref, dinv_ref, b_ref, batch_ref,
                  wh1_ref, bh1_ref, wh2_ref, bh2_ref, out_ref,
                  sums_acc, cnts_acc):
    i = pl.program_id(0)
    h = jnp.maximum(dinv_ref[...] * (s0_ref[...] + s1_ref[...] + gp_ref[...])
                    + b_ref[...], 0.0)
    gids = jax.lax.broadcasted_iota(jnp.int32, (BR, G), 1)
    onehot = (batch_ref[...] == gids).astype(jnp.float32)
    dn = (((0,), (0,)), ((), ()))
    sums = lax.dot_general(onehot, h, dn,
                           preferred_element_type=jnp.float32)
    cnts = lax.dot_general(onehot, jnp.ones((BR, D), jnp.float32), dn,
                           preferred_element_type=jnp.float32)

    @pl.when(i == 0)
    def _():
        sums_acc[...] = jnp.zeros_like(sums_acc)
        cnts_acc[...] = jnp.zeros_like(cnts_acc)

    sums_acc[...] += sums
    cnts_acc[...] += cnts

    @pl.when(i == GRID - 1)
    def _():
        ge = sums_acc[...] / jnp.maximum(cnts_acc[...], 1.0)
        hid = jnp.maximum(
            jnp.dot(ge, wh1_ref[...], preferred_element_type=jnp.float32)
            + bh1_ref[...], 0.0)
        out_ref[...] = (jnp.dot(hid, wh2_ref[...],
                                preferred_element_type=jnp.float32)
                        + bh2_ref[...])


def _tc_pool(s0, s1, gp, dinv, b, batch2d, wh1, bh1, wh2, bh2):
    return pl.pallas_call(
        _tc_pool_body,
        grid=(GRID,),
        in_specs=[
            pl.BlockSpec((BR, D), lambda i: (i, 0)),
            pl.BlockSpec((BR, D), lambda i: (i, 0)),
            pl.BlockSpec((BR, D), lambda i: (i, 0)),
            pl.BlockSpec((BR, 1), lambda i: (i, 0)),
            pl.BlockSpec((1, D), lambda i: (0, 0)),
            pl.BlockSpec((BR, 1), lambda i: (i, 0)),
            pl.BlockSpec((D, G), lambda i: (0, 0)),
            pl.BlockSpec((1, G), lambda i: (0, 0)),
            pl.BlockSpec((G, 1), lambda i: (0, 0)),
            pl.BlockSpec((1, 1), lambda i: (0, 0)),
        ],
        out_specs=pl.BlockSpec((G, 1), lambda i: (0, 0)),
        out_shape=jax.ShapeDtypeStruct((G, 1), jnp.float32),
        scratch_shapes=[
            pltpu.VMEM((G, D), jnp.float32),
            pltpu.VMEM((G, D), jnp.float32),
        ],
        compiler_params=pltpu.CompilerParams(
            dimension_semantics=("arbitrary",)),
    )(s0, s1, gp, dinv, b, batch2d, wh1, bh1, wh2, bh2)


def kernel(x, edge_index, batch, W_stack, b_stack, Wh1, bh1, Wh2, bh2):
    pad = EP - E
    src_r = jnp.concatenate(
        [edge_index[0], jnp.zeros((pad,), jnp.int32)]).reshape(NW * CH, 1, CB)
    dst_r = jnp.concatenate(
        [edge_index[1], jnp.full((pad,), N, jnp.int32)]).reshape(NW * CH, 1, CB)
    e_r = jnp.concatenate([src_r, dst_r], axis=1)
    zeros = jnp.zeros((ROWS_PER_TILE, D), jnp.float32)
    ones_tab = jnp.ones((N, D), jnp.float32)

    s_deg = _sc_agg(e_r, ones_tab, zeros)
    g, dinv = _tc_first(s_deg[0], s_deg[1], x, W_stack[0])

    for l in range(1, L):
        s_agg = _sc_agg(e_r, g, zeros)
        g = _tc_layer(s_agg[0], s_agg[1], g, dinv,
                      b_stack[l - 1].reshape(1, D), W_stack[l])

    s_agg = _sc_agg(e_r, g, zeros)
    out = _tc_pool(s_agg[0], s_agg[1], g, dinv, b_stack[L - 1].reshape(1, D),
                   batch.reshape(N, 1), Wh1, bh1.reshape(1, G), Wh2,
                   bh2.reshape(1, 1))
    return out.reshape(G)


# speedup vs baseline: 1.3626x; 1.3626x over previous
import functools

import jax
import jax.numpy as jnp
from jax import lax
from jax.experimental import pallas as pl
from jax.experimental.pallas import tpu as pltpu
from jax.experimental.pallas import tpu_sc as plsc

N = 10000
E = 320000
D = 128
L = 16
G = 64

NC = 2
NS = 16
NW = NC * NS
CB = 128
CH = 80
EP = NW * CH * CB
NBUF = 2
NI = 4
NP = 10112
ROWS_PER_TILE = NP // NS


def _sc_agg_body(src_hbm, dst_hbm, g_hbm, zeros_hbm, out_hbm,
                 s_sh, rows_v, *rest):
    src_v = rest[:NI]
    dst_v = rest[NI:2 * NI]
    gsem = rest[2 * NI:2 * NI + NBUF]
    ssem = rest[2 * NI + NBUF:2 * NI + NBUF + NI]
    dsem = rest[2 * NI + NBUF + NI:]

    c = lax.axis_index("c")
    s = lax.axis_index("s")
    w = c * NS + s

    pltpu.sync_copy(zeros_hbm, s_sh.at[pl.ds(s * ROWS_PER_TILE, ROWS_PER_TILE)])

    def idx_issue(q, j):
        pltpu.async_copy(src_hbm.at[w * CH + j], src_v[q], ssem[q])
        pltpu.async_copy(dst_hbm.at[w * CH + j], dst_v[q], dsem[q])

    def idx_wait(sem):
        pltpu.make_async_copy(src_hbm.at[0], src_v[0], sem).wait()

    def gather_issue(b, q):
        pltpu.async_copy(g_hbm.at[src_v[q].at[0]], rows_v.at[b], gsem[b])

    def gather_wait(b):
        pltpu.make_async_copy(g_hbm.at[pl.ds(0, CB)], rows_v.at[b],
                              gsem[b]).wait()

    for q in range(NI):
        idx_issue(q, q)
    for b in range(NBUF):
        idx_wait(ssem[b])
        gather_issue(b, b)

    plsc.subcore_barrier()

    def step(i, carry):
        for u in range(NI):
            j = i * NI + u
            b = u % NBUF
            q = u
            q2 = (u + NBUF) % NI
            gather_wait(b)
            idx_wait(dsem[q])
            pltpu.sync_copy(rows_v.at[b], s_sh.at[dst_v[q].at[0]], add=True)
            idx_issue(q, j + NI)
            idx_wait(ssem[q2])
            gather_issue(b, q2)
        return carry

    lax.fori_loop(0, CH // NI - 1, step, 0)

    for u in range(NI):
        b = u % NBUF
        gather_wait(b)
        idx_wait(dsem[u])
        pltpu.sync_copy(rows_v.at[b], s_sh.at[dst_v[u].at[0]], add=True)
        if u < NI - NBUF:
            q2 = (u + NBUF) % NI
            idx_wait(ssem[q2])
            gather_issue(b, q2)

    plsc.subcore_barrier()

    sl = pl.ds(s * ROWS_PER_TILE, ROWS_PER_TILE)
    pltpu.sync_copy(s_sh.at[sl], out_hbm.at[c].at[sl])


_sc_agg = functools.partial(
    pl.kernel,
    out_type=jax.ShapeDtypeStruct((NC, NP, D), jnp.float32),
    mesh=plsc.VectorSubcoreMesh(core_axis_name="c", subcore_axis_name="s",
                                num_cores=NC, num_subcores=NS),
    scratch_types=[
        pltpu.VMEM_SHARED((NP, D), jnp.float32),
        pltpu.VMEM((NBUF, CB, D), jnp.float32),
    ]
    + [pltpu.VMEM((1, CB), jnp.int32) for _ in range(2 * NI)]
    + [pltpu.SemaphoreType.DMA] * (NBUF + 2 * NI),
)(_sc_agg_body)


def _sc_deg_body(dst_hbm, ones_hbm, zeros_hbm, out_hbm,
                 s_sh, ones_v, *rest):
    dst_v = rest[:NI]
    dsem = rest[NI:]

    c = lax.axis_index("c")
    s = lax.axis_index("s")
    w = c * NS + s

    pltpu.sync_copy(zeros_hbm, s_sh.at[pl.ds(s * ROWS_PER_TILE, ROWS_PER_TILE)])
    pltpu.sync_copy(ones_hbm, ones_v)

    def idx_issue(q, j):
        pltpu.async_copy(dst_hbm.at[w * CH + j], dst_v[q], dsem[q])

    def idx_wait(q):
        pltpu.make_async_copy(dst_hbm.at[0], dst_v[0], dsem[q]).wait()

    for q in range(NI):
        idx_issue(q, q)

    plsc.subcore_barrier()

    def step(i, carry):
        for u in range(NI):
            j = i * NI + u
            idx_wait(u)
            pltpu.sync_copy(ones_v, s_sh.at[dst_v[u].at[0]], add=True)
            idx_issue(u, j + NI)
        return carry

    lax.fori_loop(0, CH // NI - 1, step, 0)

    for u in range(NI):
        idx_wait(u)
        pltpu.sync_copy(ones_v, s_sh.at[dst_v[u].at[0]], add=True)

    plsc.subcore_barrier()

    sl = pl.ds(s * ROWS_PER_TILE, ROWS_PER_TILE)
    pltpu.sync_copy(s_sh.at[sl], out_hbm.at[c].at[sl])


_sc_deg = functools.partial(
    pl.kernel,
    out_type=jax.ShapeDtypeStruct((NC, NP, D), jnp.float32),
    mesh=plsc.VectorSubcoreMesh(core_axis_name="c", subcore_axis_name="s",
                                num_cores=NC, num_subcores=NS),
    scratch_types=[
        pltpu.VMEM_SHARED((NP, D), jnp.float32),
        pltpu.VMEM((CB, D), jnp.float32),
    ]
    + [pltpu.VMEM((1, CB), jnp.int32) for _ in range(NI)]
    + [pltpu.SemaphoreType.DMA] * NI,
)(_sc_deg_body)


BR = 1000
GRID = N // BR


def _tc_first_body(s0_ref, s1_ref, x_ref, w_ref, g_ref, dinv_ref):
    deg = s0_ref[:, 0:1] + s1_ref[:, 0:1] + 1.0
    dinv = lax.rsqrt(deg)
    dinv_ref[...] = dinv
    xw = jnp.dot(x_ref[...], w_ref[...], preferred_element_type=jnp.float32)
    g_ref[...] = dinv * xw


def _tc_first(s0, s1, x, w0):
    return pl.pallas_call(
        _tc_first_body,
        grid=(GRID,),
        in_specs=[
            pl.BlockSpec((BR, D), lambda i: (i, 0)),
            pl.BlockSpec((BR, D), lambda i: (i, 0)),
            pl.BlockSpec((BR, D), lambda i: (i, 0)),
            pl.BlockSpec((D, D), lambda i: (0, 0)),
        ],
        out_specs=[
            pl.BlockSpec((BR, D), lambda i: (i, 0)),
            pl.BlockSpec((BR, 1), lambda i: (i, 0)),
        ],
        out_shape=[
            jax.ShapeDtypeStruct((N, D), jnp.float32),
            jax.ShapeDtypeStruct((N, 1), jnp.float32),
        ],
    )(s0, s1, x, w0)


def _tc_layer_body(s0_ref, s1_ref, gp_ref, dinv_ref, b_ref, w_ref, g_ref):
    dinv = dinv_ref[...]
    h = jnp.maximum(dinv * (s0_ref[...] + s1_ref[...] + gp_ref[...])
                    + b_ref[...], 0.0)
    hw = jnp.dot(h, w_ref[...], preferred_element_type=jnp.float32)
    g_ref[...] = dinv * hw


def _tc_layer(s0, s1, gp, dinv, b, w):
    return pl.pallas_call(
        _tc_layer_body,
        grid=(GRID,),
        in_specs=[
            pl.BlockSpec((BR, D), lambda i: (i, 0)),
            pl.BlockSpec((BR, D), lambda i: (i, 0)),
            pl.BlockSpec((BR, D), lambda i: (i, 0)),
            pl.BlockSpec((BR, 1), lambda i: (i, 0)),
            pl.BlockSpec((1, D), lambda i: (0, 0)),
            pl.BlockSpec((D, D), lambda i: (0, 0)),
        ],
        out_specs=pl.BlockSpec((BR, D), lambda i: (i, 0)),
        out_shape=jax.ShapeDtypeStruct((N, D), jnp.float32),
    )(s0, s1, gp, dinv, b, w)


def _tc_pool_body(s0_ref, s1_ref, gp_ref, dinv_ref, b_ref, batch_ref,
                  wh1_ref, bh1_ref, wh2_ref, bh2_ref, out_ref,
                  sums_acc, cnts_acc):
    i = pl.program_id(0)
    h = jnp.maximum(dinv_ref[...] * (s0_ref[...] + s1_ref[...] + gp_ref[...])
                    + b_ref[...], 0.0)
    gids = jax.lax.broadcasted_iota(jnp.int32, (BR, G), 1)
    onehot = (batch_ref[...] == gids).astype(jnp.float32)
    dn = (((0,), (0,)), ((), ()))
    sums = lax.dot_general(onehot, h, dn,
                           preferred_element_type=jnp.float32)
    cnts = lax.dot_general(onehot, jnp.ones((BR, D), jnp.float32), dn,
                           preferred_element_type=jnp.float32)

    @pl.when(i == 0)
    def _():
        sums_acc[...] = jnp.zeros_like(sums_acc)
        cnts_acc[...] = jnp.zeros_like(cnts_acc)

    sums_acc[...] += sums
    cnts_acc[...] += cnts

    @pl.when(i == GRID - 1)
    def _():
        ge = sums_acc[...] / jnp.maximum(cnts_acc[...], 1.0)
        hid = jnp.maximum(
            jnp.dot(ge, wh1_ref[...], preferred_element_type=jnp.float32)
            + bh1_ref[...], 0.0)
        out_ref[...] = (jnp.dot(hid, wh2_ref[...],
                                preferred_element_type=jnp.float32)
                        + bh2_ref[...])


def _tc_pool(s0, s1, gp, dinv, b, batch2d, wh1, bh1, wh2, bh2):
    return pl.pallas_call(
        _tc_pool_body,
        grid=(GRID,),
        in_specs=[
            pl.BlockSpec((BR, D), lambda i: (i, 0)),
            pl.BlockSpec((BR, D), lambda i: (i, 0)),
            pl.BlockSpec((BR, D), lambda i: (i, 0)),
            pl.BlockSpec((BR, 1), lambda i: (i, 0)),
            pl.BlockSpec((1, D), lambda i: (0, 0)),
            pl.BlockSpec((BR, 1), lambda i: (i, 0)),
            pl.BlockSpec((D, G), lambda i: (0, 0)),
            pl.BlockSpec((1, G), lambda i: (0, 0)),
            pl.BlockSpec((G, 1), lambda i: (0, 0)),
            pl.BlockSpec((1, 1), lambda i: (0, 0)),
        ],
        out_specs=pl.BlockSpec((G, 1), lambda i: (0, 0)),
        out_shape=jax.ShapeDtypeStruct((G, 1), jnp.float32),
        scratch_shapes=[
            pltpu.VMEM((G, D), jnp.float32),
            pltpu.VMEM((G, D), jnp.float32),
        ],
        compiler_params=pltpu.CompilerParams(
            dimension_semantics=("arbitrary",)),
    )(s0, s1, gp, dinv, b, batch2d, wh1, bh1, wh2, bh2)


def kernel(x, edge_index, batch, W_stack, b_stack, Wh1, bh1, Wh2, bh2):
    pad = EP - E
    src_r = jnp.concatenate(
        [edge_index[0], jnp.zeros((pad,), jnp.int32)]).reshape(NW * CH, 1, CB)
    dst_r = jnp.concatenate(
        [edge_index[1], jnp.full((pad,), N, jnp.int32)]).reshape(NW * CH, 1, CB)
    zeros = jnp.zeros((ROWS_PER_TILE, D), jnp.float32)
    ones_blk = jnp.ones((CB, D), jnp.float32)

    s_deg = _sc_deg(dst_r, ones_blk, zeros)
    g, dinv = _tc_first(s_deg[0], s_deg[1], x, W_stack[0])

    for l in range(1, L):
        s_agg = _sc_agg(src_r, dst_r, g, zeros)
        g = _tc_layer(s_agg[0], s_agg[1], g, dinv,
                      b_stack[l - 1].reshape(1, D), W_stack[l])

    s_agg = _sc_agg(src_r, dst_r, g, zeros)
    out = _tc_pool(s_agg[0], s_agg[1], g, dinv, b_stack[L - 1].reshape(1, D),
                   batch.reshape(N, 1), Wh1, bh1.reshape(1, G), Wh2,
                   bh2.reshape(1, 1))
    return out.reshape(G)
